# Initial kernel scaffold; baseline (speedup 1.0000x reference)
import jax
import jax.numpy as jnp
from jax.experimental import pallas as pl


def kernel(type_ids, c, gm, pos, r, vid, edge_index, batch, W1, b1, W2, b2, Wg, bg, Wo, bo):
    raise NotImplementedError("write your pallas kernel here")


# speedup vs baseline: 9.7680x; 9.7680x over previous
import dataclasses
import functools

import jax
import jax.numpy as jnp
from jax import lax
from jax.experimental import pallas as pl
from jax.experimental.pallas import tpu as pltpu
from jax.experimental.pallas import tpu_sc as plsc

N = 100000
E = 1600000
F = 32
NUM_TYPES = 25
NUM_GRAPHS = 256
HALF = N // 2

BLK = 2000
NBLK = N // BLK

SUB = 128
SUPER = 1024
N_SUPER = (E + SUPER - 1) // SUPER
SUPER_PER_TILE = (N_SUPER + 15) // 16
PAD_E = 1601536
EROWS = PAD_E // SUB
SUPER2 = 4096
N_SUPER2 = (E + SUPER2 - 1) // SUPER2
SUPER2_PER_TILE = (N_SUPER2 + 15) // 16

AGG_ROWS = 51200
TRASH0 = 50048
ZSTRIPE = AGG_ROWS // 16
WSTRIPE = HALF // 16
HSTRIPE = HALF // 16


def _feat_body(tid_ref, feats_ref, wc_ref, bc_ref, z_ref):
    t = tid_ref[0, 0, :]
    oh = (t[:, None] == lax.broadcasted_iota(jnp.int32, (BLK, NUM_TYPES), 1))
    f = jnp.concatenate([oh.astype(jnp.float32), feats_ref[...]], axis=1)
    z_ref[...] = (
        jnp.dot(f, wc_ref[...], preferred_element_type=jnp.float32)
        + bc_ref[...]
    )


def _deg_body(dst_hbm, deg_hbm, dst_st, hist):
    c = lax.axis_index("c")
    s = lax.axis_index("s")
    cbase = c * HALF
    ones16 = jnp.ones((16,), jnp.float32)

    @pl.loop(0, HALF, step=16)
    def _(i):
        hist[pl.ds(i, 16)] = jnp.zeros((16,), jnp.float32)

    @pl.loop(0, SUPER2_PER_TILE)
    def _(it):
        sup = s + 16 * it

        @pl.when(sup < N_SUPER2)
        def _():
            row0 = pl.multiple_of(sup * (SUPER2 // SUB), 8)
            pltpu.sync_copy(dst_hbm.at[pl.ds(row0, SUPER2 // SUB)], dst_st)
            base = sup * SUPER2
            for j in range(SUPER2 // SUB):
                @pl.when(base + j * SUB < E)
                def _(j=j):
                    for v in range(SUB // 16):
                        d = dst_st[j, pl.ds(v * 16, 16)]
                        dl = d - cbase
                        ok = (dl >= 0) & (dl < HALF)
                        plsc.addupdate_scatter(
                            hist, [jnp.where(ok, dl, 0)], ones16, mask=ok)

    for k in range(HALF // BLK):
        pltpu.sync_copy(hist.at[pl.ds(k * BLK, BLK)],
                        deg_hbm.at[c * (HALF // BLK) + k, s, 0])


def _edge_body(z_hbm, src_hbm, dst_hbm, agg_hbm,
               agg_sh, src_st, dst_st, rows, dloc, zeros_blk,
               sem_a, sem_b):
    c = lax.axis_index("c")
    s = lax.axis_index("s")
    cbase = c * HALF

    @pl.loop(0, SUB)
    def _(r):
        zeros_blk[r, pl.ds(0, 16)] = jnp.zeros((16,), jnp.float32)
        zeros_blk[r, pl.ds(16, 16)] = jnp.zeros((16,), jnp.float32)

    zoff = pl.multiple_of(s * ZSTRIPE, 8)
    for k in range(ZSTRIPE // SUB):
        pltpu.sync_copy(zeros_blk,
                        agg_sh.at[pl.ds(zoff + k * SUB, SUB)])

    plsc.subcore_barrier()

    iota16 = lax.iota(jnp.int32, 16)

    @pl.loop(0, SUPER_PER_TILE)
    def _(it):
        sup = s + 16 * it
        row0 = pl.multiple_of(sup * (SUPER // SUB), 8)

        @pl.when(sup < N_SUPER)
        def _():
            pltpu.sync_copy(src_hbm.at[pl.ds(row0, SUPER // SUB)], src_st)
            pltpu.sync_copy(dst_hbm.at[pl.ds(row0, SUPER // SUB)], dst_st)

            base = sup * SUPER
            sems = (sem_a, sem_b)

            def gather_args(j):
                return z_hbm.at[src_st.at[j]], rows.at[j % 2], sems[j % 2]

            pltpu.async_copy(*gather_args(0))
            for j in range(SUPER // SUB):
                valid_j = base + j * SUB < E

                @pl.when(valid_j)
                def _(j=j):
                    pltpu.make_async_copy(*gather_args(j)).wait()
                    for v in range(SUB // 16):
                        d = dst_st[j, pl.ds(v * 16, 16)]
                        dl = d - cbase
                        ok = (dl >= 0) & (dl < HALF)
                        tr = TRASH0 + ((iota16 + (sup * 29 + j * 8 + v))
                                       & 1023)
                        dloc[pl.ds(v * 16, 16)] = jnp.where(ok, dl, tr)

                if j + 1 < SUPER // SUB:
                    @pl.when(base + (j + 1) * SUB < E)
                    def _(j=j):
                        pltpu.async_copy(*gather_args(j + 1))

                @pl.when(valid_j)
                def _(j=j):
                    pltpu.sync_copy(rows.at[j % 2], agg_sh.at[dloc],
                                    add=True)

    plsc.subcore_barrier()

    @pl.when(s == 0)
    def _():
        cb = pl.multiple_of(cbase, 8)
        pltpu.sync_copy(agg_sh.at[pl.ds(0, HALF)],
                        agg_hbm.at[pl.ds(cb, HALF)])


def _post_body(agg_ref, deg_ref, batch_ref, wg_ref, bg_ref, wo_ref, bo_ref,
               out_ref, acc_ref):
    i = pl.program_id(0)
    deg = jnp.maximum(jnp.sum(deg_ref[0, :, 0, :], axis=0), 1.0)
    h = agg_ref[...] / deg[:, None]
    h = jnp.maximum(
        jnp.dot(h, wg_ref[...], preferred_element_type=jnp.float32)
        + bg_ref[...], 0.0)
    b = batch_ref[0, 0, :]
    oh = (b[:, None] == lax.broadcasted_iota(jnp.int32, (BLK, NUM_GRAPHS), 1))
    hx = jnp.concatenate([h, jnp.ones((BLK, 1), jnp.float32)], axis=1)
    part = lax.dot_general(oh.astype(jnp.float32), hx,
                           (((0,), (0,)), ((), ())),
                           preferred_element_type=jnp.float32)

    @pl.when(i == 0)
    def _():
        acc_ref[...] = part

    @pl.when(i > 0)
    def _():
        acc_ref[...] += part

    @pl.when(i == NBLK - 1)
    def _():
        sums = acc_ref[:, :F]
        cnt = jnp.maximum(acc_ref[:, F:F + 1], 1.0)
        out_ref[...] = (
            jnp.dot(sums / cnt, wo_ref[...],
                    preferred_element_type=jnp.float32)
            + bo_ref[...])


def kernel(type_ids, c, gm, pos, r, vid, edge_index, batch,
           W1, b1, W2, b2, Wg, bg, Wo, bo):
    tid3 = type_ids.astype(jnp.int32).reshape(NBLK, 1, BLK)
    feats = jnp.stack([c, gm, pos, r, vid], axis=-1).astype(jnp.float32)
    wc = jnp.zeros((NUM_TYPES + 5, F), jnp.float32)
    wc = wc.at[:NUM_TYPES, :F // 2].set(W1).at[NUM_TYPES:, F // 2:].set(W2)
    bc = jnp.concatenate([b1, b2]).reshape(1, F)

    src = jnp.pad(edge_index[0].astype(jnp.int32), (0, PAD_E - E))
    dst = jnp.pad(edge_index[1].astype(jnp.int32), (0, PAD_E - E))
    src2 = src.reshape(EROWS, SUB)
    dst2 = dst.reshape(EROWS, SUB)

    z = pl.pallas_call(
        _feat_body,
        grid=(NBLK,),
        in_specs=[
            pl.BlockSpec((1, 1, BLK), lambda i: (i, 0, 0)),
            pl.BlockSpec((BLK, 5), lambda i: (i, 0)),
            pl.BlockSpec((NUM_TYPES + 5, F), lambda i: (0, 0)),
            pl.BlockSpec((1, F), lambda i: (0, 0)),
        ],
        out_specs=pl.BlockSpec((BLK, F), lambda i: (i, 0)),
        out_shape=jax.ShapeDtypeStruct((N, F), jnp.float32),
    )(tid3, feats, wc, bc)

    mesh = plsc.VectorSubcoreMesh(core_axis_name="c", subcore_axis_name="s")
    cp = pltpu.CompilerParams()
    if "needs_layout_passes" in pltpu.CompilerParams.__dataclass_fields__:
        cp = dataclasses.replace(cp, needs_layout_passes=False)
    if "use_tc_tiling_on_sc" in pltpu.CompilerParams.__dataclass_fields__:
        cp = dataclasses.replace(cp, use_tc_tiling_on_sc=False)
    deg16 = pl.kernel(
        _deg_body,
        out_type=jax.ShapeDtypeStruct((NBLK, 16, 1, BLK), jnp.float32),
        mesh=mesh,
        scratch_types=[
            pltpu.VMEM((SUPER2 // SUB, SUB), jnp.int32),
            pltpu.VMEM((HALF,), jnp.float32),
        ],
        compiler_params=cp,
    )(dst2)

    agg = pl.kernel(
        _edge_body,
        out_type=jax.ShapeDtypeStruct((N, F), jnp.float32),
        mesh=mesh,
        scratch_types=[
            pltpu.VMEM_SHARED((AGG_ROWS, F), jnp.float32),
            pltpu.VMEM((SUPER // SUB, SUB), jnp.int32),
            pltpu.VMEM((SUPER // SUB, SUB), jnp.int32),
            pltpu.VMEM((2, SUB, F), jnp.float32),
            pltpu.VMEM((SUB,), jnp.int32),
            pltpu.VMEM((SUB, F), jnp.float32),
            pltpu.SemaphoreType.DMA,
            pltpu.SemaphoreType.DMA,
        ],
        compiler_params=cp,
    )(z, src2, dst2)

    batch3 = batch.astype(jnp.int32).reshape(NBLK, 1, BLK)
    pred = pl.pallas_call(
        _post_body,
        grid=(NBLK,),
        in_specs=[
            pl.BlockSpec((BLK, F), lambda i: (i, 0)),
            pl.BlockSpec((1, 16, 1, BLK), lambda i: (i, 0, 0, 0)),
            pl.BlockSpec((1, 1, BLK), lambda i: (i, 0, 0)),
            pl.BlockSpec((F, F), lambda i: (0, 0)),
            pl.BlockSpec((1, F), lambda i: (0, 0)),
            pl.BlockSpec((F, 4), lambda i: (0, 0)),
            pl.BlockSpec((1, 4), lambda i: (0, 0)),
        ],
        out_specs=pl.BlockSpec((NUM_GRAPHS, 4), lambda i: (0, 0)),
        out_shape=jax.ShapeDtypeStruct((NUM_GRAPHS, 4), jnp.float32),
        scratch_shapes=[pltpu.VMEM((NUM_GRAPHS, F + 1), jnp.float32)],
    )(agg, deg16, batch3, Wg, bg.reshape(1, F), Wo, bo.reshape(1, 4))

    return pred


# speedup vs baseline: 11.8196x; 1.2100x over previous
import dataclasses
import functools

import jax
import jax.numpy as jnp
from jax import lax
from jax.experimental import pallas as pl
from jax.experimental.pallas import tpu as pltpu
from jax.experimental.pallas import tpu_sc as plsc

N = 100000
E = 1600000
F = 32
NUM_TYPES = 25
NUM_GRAPHS = 256
HALF = N // 2

BLK = 2000
NBLK = N // BLK

SUB = 128
SUPER = 1024
N_SUPER = (E + SUPER - 1) // SUPER
SUPER_PER_TILE = (N_SUPER + 15) // 16
PAD_E = 1605632
EROWS = PAD_E // SUB
CAP = 1280
SUPER2 = 4096
N_SUPER2 = (E + SUPER2 - 1) // SUPER2
SUPER2_PER_TILE = (N_SUPER2 + 15) // 16

AGG_ROWS = 51200
TRASH0 = 50048
ZSTRIPE = AGG_ROWS // 16
WSTRIPE = HALF // 16
HSTRIPE = HALF // 16


def _feat_body(tid_ref, feats_ref, wc_ref, bc_ref, z_ref):
    t = tid_ref[0, 0, :]
    oh = (t[:, None] == lax.broadcasted_iota(jnp.int32, (BLK, NUM_TYPES), 1))
    f = jnp.concatenate([oh.astype(jnp.float32), feats_ref[...]], axis=1)
    z_ref[...] = (
        jnp.dot(f, wc_ref[...], preferred_element_type=jnp.float32)
        + bc_ref[...]
    )


def _deg_body(dst_hbm, deg_hbm, dst_st, hist):
    c = lax.axis_index("c")
    s = lax.axis_index("s")
    cbase = c * HALF
    ones16 = jnp.ones((16,), jnp.float32)

    @pl.loop(0, HALF, step=16)
    def _(i):
        hist[pl.ds(i, 16)] = jnp.zeros((16,), jnp.float32)

    @pl.loop(0, SUPER2_PER_TILE)
    def _(it):
        sup = s + 16 * it

        @pl.when(sup < N_SUPER2)
        def _():
            row0 = pl.multiple_of(sup * (SUPER2 // SUB), 8)
            pltpu.sync_copy(dst_hbm.at[pl.ds(row0, SUPER2 // SUB)], dst_st)
            base = sup * SUPER2
            for j in range(SUPER2 // SUB):
                @pl.when(base + j * SUB < E)
                def _(j=j):
                    for v in range(SUB // 16):
                        d = dst_st[j, pl.ds(v * 16, 16)]
                        dl = d - cbase
                        ok = (dl >= 0) & (dl < HALF)
                        plsc.addupdate_scatter(
                            hist, [jnp.where(ok, dl, 0)], ones16, mask=ok)

    for k in range(HALF // BLK):
        pltpu.sync_copy(hist.at[pl.ds(k * BLK, BLK)],
                        deg_hbm.at[c * (HALF // BLK) + k, s, 0])


def _edge_body(z_hbm, src_hbm, dst_hbm, agg_hbm,
               agg_sh, src_st, dst_st, rows, cb_src, cb_dl, gsrc, gdl,
               zeros_blk, sem_a, sem_b):
    c = lax.axis_index("c")
    s = lax.axis_index("s")
    cbase = c * HALF

    @pl.loop(0, SUB)
    def _(r):
        zeros_blk[r, pl.ds(0, 16)] = jnp.zeros((16,), jnp.float32)
        zeros_blk[r, pl.ds(16, 16)] = jnp.zeros((16,), jnp.float32)

    zoff = pl.multiple_of(s * ZSTRIPE, 8)
    for k in range(ZSTRIPE // SUB):
        pltpu.sync_copy(zeros_blk,
                        agg_sh.at[pl.ds(zoff + k * SUB, SUB)])

    plsc.subcore_barrier()

    iota16 = lax.iota(jnp.int32, 16)
    sems = (sem_a, sem_b)

    def prep_slot(t, b):
        for q in range(8):
            gsrc[b, pl.ds(q * 16, 16)] = cb_src[pl.ds(t * SUB + q * 16, 16)]
            gdl[b, pl.ds(q * 16, 16)] = cb_dl[pl.ds(t * SUB + q * 16, 16)]

    def gather_args(t):
        b = t % 2
        return z_hbm.at[gsrc.at[b]], rows.at[b], sems[b]

    def _main(it, rem):
        sup = s + 16 * it
        row0 = pl.multiple_of(sup * (SUPER // SUB), 8)
        pltpu.sync_copy(src_hbm.at[pl.ds(row0, SUPER // SUB)], src_st)
        pltpu.sync_copy(dst_hbm.at[pl.ds(row0, SUPER // SUB)], dst_st)
        base = sup * SUPER
        woff = rem
        for j in range(SUPER // SUB):
            for v in range(SUB // 16):
                off = j * SUB + v * 16
                d = dst_st[j, pl.ds(v * 16, 16)]
                dl = d - cbase
                ok = (dl >= 0) & (dl < HALF) & (base + off + iota16 < E)
                sv = src_st[j, pl.ds(v * 16, 16)]
                cum = plsc.cumsum(ok.astype(jnp.int32))
                pos = woff + cum - 1
                plsc.store_scatter(cb_dl, [pos], dl, mask=ok)
                plsc.store_scatter(cb_src, [pos], sv, mask=ok)
                woff = woff + jnp.max(cum)
        nfull = lax.div(woff, SUB)

        @pl.when(nfull > 0)
        def _():
            prep_slot(0, 0)
            pltpu.async_copy(*gather_args(0))

        for t in range(8):
            @pl.when(t < nfull)
            def _(t=t):
                pltpu.make_async_copy(*gather_args(t)).wait()

            if t + 1 < 8:
                @pl.when(t + 1 < nfull)
                def _(t=t):
                    prep_slot(t + 1, (t + 1) % 2)
                    pltpu.async_copy(*gather_args(t + 1))

            @pl.when(t < nfull)
            def _(t=t):
                pltpu.sync_copy(rows.at[t % 2], agg_sh.at[gdl.at[t % 2]],
                                add=True)

        shoff = pl.multiple_of(nfull * SUB, 8)
        for q in range(8):
            t1 = cb_src[pl.ds(shoff + q * 16, 16)]
            t2 = cb_dl[pl.ds(shoff + q * 16, 16)]
            cb_src[pl.ds(q * 16, 16)] = t1
            cb_dl[pl.ds(q * 16, 16)] = t2
        return woff - nfull * SUB

    rem = pl.loop(0, SUPER_PER_TILE, init_carry=jnp.int32(0))(_main)

    for q in range(8):
        cb_src[pl.ds(rem + q * 16, 16)] = iota16 + (q * 16)
        cb_dl[pl.ds(rem + q * 16, 16)] = TRASH0 + iota16 + (q * 16)
    prep_slot(0, 0)
    pltpu.async_copy(*gather_args(0))
    pltpu.make_async_copy(*gather_args(0)).wait()
    pltpu.sync_copy(rows.at[0], agg_sh.at[gdl.at[0]], add=True)

    plsc.subcore_barrier()

    @pl.when(s == 0)
    def _():
        cb = pl.multiple_of(cbase, 8)
        pltpu.sync_copy(agg_sh.at[pl.ds(0, HALF)],
                        agg_hbm.at[pl.ds(cb, HALF)])


def _post_body(agg_ref, deg_ref, batch_ref, wg_ref, bg_ref, wo_ref, bo_ref,
               out_ref, acc_ref):
    i = pl.program_id(0)
    deg = jnp.maximum(jnp.sum(deg_ref[0, :, 0, :], axis=0), 1.0)
    h = agg_ref[...] / deg[:, None]
    h = jnp.maximum(
        jnp.dot(h, wg_ref[...], preferred_element_type=jnp.float32)
        + bg_ref[...], 0.0)
    b = batch_ref[0, 0, :]
    oh = (b[:, None] == lax.broadcasted_iota(jnp.int32, (BLK, NUM_GRAPHS), 1))
    hx = jnp.concatenate([h, jnp.ones((BLK, 1), jnp.float32)], axis=1)
    part = lax.dot_general(oh.astype(jnp.float32), hx,
                           (((0,), (0,)), ((), ())),
                           preferred_element_type=jnp.float32)

    @pl.when(i == 0)
    def _():
        acc_ref[...] = part

    @pl.when(i > 0)
    def _():
        acc_ref[...] += part

    @pl.when(i == NBLK - 1)
    def _():
        sums = acc_ref[:, :F]
        cnt = jnp.maximum(acc_ref[:, F:F + 1], 1.0)
        out_ref[...] = (
            jnp.dot(sums / cnt, wo_ref[...],
                    preferred_element_type=jnp.float32)
            + bo_ref[...])


def kernel(type_ids, c, gm, pos, r, vid, edge_index, batch,
           W1, b1, W2, b2, Wg, bg, Wo, bo):
    tid3 = type_ids.astype(jnp.int32).reshape(NBLK, 1, BLK)
    feats = jnp.stack([c, gm, pos, r, vid], axis=-1).astype(jnp.float32)
    wc = jnp.zeros((NUM_TYPES + 5, F), jnp.float32)
    wc = wc.at[:NUM_TYPES, :F // 2].set(W1).at[NUM_TYPES:, F // 2:].set(W2)
    bc = jnp.concatenate([b1, b2]).reshape(1, F)

    src = jnp.pad(edge_index[0].astype(jnp.int32), (0, PAD_E - E))
    dst = jnp.pad(edge_index[1].astype(jnp.int32), (0, PAD_E - E))
    src2 = src.reshape(EROWS, SUB)
    dst2 = dst.reshape(EROWS, SUB)

    z = pl.pallas_call(
        _feat_body,
        grid=(NBLK,),
        in_specs=[
            pl.BlockSpec((1, 1, BLK), lambda i: (i, 0, 0)),
            pl.BlockSpec((BLK, 5), lambda i: (i, 0)),
            pl.BlockSpec((NUM_TYPES + 5, F), lambda i: (0, 0)),
            pl.BlockSpec((1, F), lambda i: (0, 0)),
        ],
        out_specs=pl.BlockSpec((BLK, F), lambda i: (i, 0)),
        out_shape=jax.ShapeDtypeStruct((N, F), jnp.float32),
    )(tid3, feats, wc, bc)

    mesh = plsc.VectorSubcoreMesh(core_axis_name="c", subcore_axis_name="s")
    cp = pltpu.CompilerParams()
    if "needs_layout_passes" in pltpu.CompilerParams.__dataclass_fields__:
        cp = dataclasses.replace(cp, needs_layout_passes=False)
    if "use_tc_tiling_on_sc" in pltpu.CompilerParams.__dataclass_fields__:
        cp = dataclasses.replace(cp, use_tc_tiling_on_sc=False)
    deg16 = pl.kernel(
        _deg_body,
        out_type=jax.ShapeDtypeStruct((NBLK, 16, 1, BLK), jnp.float32),
        mesh=mesh,
        scratch_types=[
            pltpu.VMEM((SUPER2 // SUB, SUB), jnp.int32),
            pltpu.VMEM((HALF,), jnp.float32),
        ],
        compiler_params=cp,
    )(dst2)

    agg = pl.kernel(
        _edge_body,
        out_type=jax.ShapeDtypeStruct((N, F), jnp.float32),
        mesh=mesh,
        scratch_types=[
            pltpu.VMEM_SHARED((AGG_ROWS, F), jnp.float32),
            pltpu.VMEM((SUPER // SUB, SUB), jnp.int32),
            pltpu.VMEM((SUPER // SUB, SUB), jnp.int32),
            pltpu.VMEM((2, SUB, F), jnp.float32),
            pltpu.VMEM((CAP,), jnp.int32),
            pltpu.VMEM((CAP,), jnp.int32),
            pltpu.VMEM((2, SUB), jnp.int32),
            pltpu.VMEM((2, SUB), jnp.int32),
            pltpu.VMEM((SUB, F), jnp.float32),
            pltpu.SemaphoreType.DMA,
            pltpu.SemaphoreType.DMA,
        ],
        compiler_params=cp,
    )(z, src2, dst2)

    batch3 = batch.astype(jnp.int32).reshape(NBLK, 1, BLK)
    pred = pl.pallas_call(
        _post_body,
        grid=(NBLK,),
        in_specs=[
            pl.BlockSpec((BLK, F), lambda i: (i, 0)),
            pl.BlockSpec((1, 16, 1, BLK), lambda i: (i, 0, 0, 0)),
            pl.BlockSpec((1, 1, BLK), lambda i: (i, 0, 0)),
            pl.BlockSpec((F, F), lambda i: (0, 0)),
            pl.BlockSpec((1, F), lambda i: (0, 0)),
            pl.BlockSpec((F, 4), lambda i: (0, 0)),
            pl.BlockSpec((1, 4), lambda i: (0, 0)),
        ],
        out_specs=pl.BlockSpec((NUM_GRAPHS, 4), lambda i: (0, 0)),
        out_shape=jax.ShapeDtypeStruct((NUM_GRAPHS, 4), jnp.float32),
        scratch_shapes=[pltpu.VMEM((NUM_GRAPHS, F + 1), jnp.float32)],
    )(agg, deg16, batch3, Wg, bg.reshape(1, F), Wo, bo.reshape(1, 4))

    return pred


# speedup vs baseline: 12.7007x; 1.0746x over previous
import dataclasses
import functools

import jax
import jax.numpy as jnp
from jax import lax
from jax.experimental import pallas as pl
from jax.experimental.pallas import tpu as pltpu
from jax.experimental.pallas import tpu_sc as plsc

N = 100000
E = 1600000
F = 32
NUM_TYPES = 25
NUM_GRAPHS = 256
HALF = N // 2

BLK = 2000
NBLK = N // BLK
BLK_A = 10000
NBLK_A = N // BLK_A

SUB = 128
SUPER = 1024
N_SUPER = (E + SUPER - 1) // SUPER
SUPER_PER_TILE = (N_SUPER + 15) // 16
PAD_E = 1605632
EROWS = PAD_E // SUB
CAP = 1280
SUPER2 = 4096
N_SUPER2 = (E + SUPER2 - 1) // SUPER2
SUPER2_PER_TILE = (N_SUPER2 + 15) // 16

AGG_ROWS = 51200
TRASH0 = 50048
ZSTRIPE = AGG_ROWS // 16
WSTRIPE = HALF // 16
HSTRIPE = HALF // 16


def _feat_body(tid_ref, c_ref, gm_ref, pos_ref, r_ref, vid_ref,
               w1_ref, b1_ref, w2_ref, b2_ref, z_ref):
    t = tid_ref[0, 0, :]
    oh = (t[:, None] == lax.broadcasted_iota(jnp.int32, (BLK_A, NUM_TYPES), 1))
    xt = (jnp.dot(oh.astype(jnp.float32), w1_ref[...],
                  preferred_element_type=jnp.float32) + b1_ref[...])
    w2 = w2_ref[...]
    xn = (b2_ref[...]
          + c_ref[0, 0, :][:, None] * w2[0:1, :]
          + gm_ref[0, 0, :][:, None] * w2[1:2, :]
          + pos_ref[0, 0, :][:, None] * w2[2:3, :]
          + r_ref[0, 0, :][:, None] * w2[3:4, :]
          + vid_ref[0, 0, :][:, None] * w2[4:5, :])
    z_ref[...] = jnp.concatenate([xt, xn], axis=1)


def _deg_body(dst_hbm, deg_hbm, dst_st, hist):
    c = lax.axis_index("c")
    s = lax.axis_index("s")
    cbase = c * HALF
    ones16 = jnp.ones((16,), jnp.float32)

    @pl.loop(0, HALF, step=16)
    def _(i):
        hist[pl.ds(i, 16)] = jnp.zeros((16,), jnp.float32)

    @pl.loop(0, SUPER2_PER_TILE)
    def _(it):
        sup = s + 16 * it

        @pl.when(sup < N_SUPER2)
        def _():
            row0 = pl.multiple_of(sup * (SUPER2 // SUB), 8)
            pltpu.sync_copy(dst_hbm.at[pl.ds(row0, SUPER2 // SUB)], dst_st)
            base = sup * SUPER2
            for j in range(SUPER2 // SUB):
                @pl.when(base + j * SUB < E)
                def _(j=j):
                    for v in range(SUB // 16):
                        d = dst_st[j, pl.ds(v * 16, 16)]
                        dl = d - cbase
                        ok = (dl >= 0) & (dl < HALF)
                        plsc.addupdate_scatter(
                            hist, [jnp.where(ok, dl, 0)], ones16, mask=ok)

    for k in range(HALF // BLK):
        pltpu.sync_copy(hist.at[pl.ds(k * BLK, BLK)],
                        deg_hbm.at[c * (HALF // BLK) + k, s, 0])


def _edge_body(z_hbm, src_hbm, dst_hbm, agg_hbm,
               agg_sh, src_st, dst_st, rows, cb_src, cb_dl, gsrc, gdl,
               zeros_blk, sem_a, sem_b):
    c = lax.axis_index("c")
    s = lax.axis_index("s")
    cbase = c * HALF

    @pl.loop(0, SUB)
    def _(r):
        zeros_blk[r, pl.ds(0, 16)] = jnp.zeros((16,), jnp.float32)
        zeros_blk[r, pl.ds(16, 16)] = jnp.zeros((16,), jnp.float32)

    zoff = pl.multiple_of(s * ZSTRIPE, 8)
    for k in range(ZSTRIPE // SUB):
        pltpu.sync_copy(zeros_blk,
                        agg_sh.at[pl.ds(zoff + k * SUB, SUB)])

    plsc.subcore_barrier()

    iota16 = lax.iota(jnp.int32, 16)
    sems = (sem_a, sem_b)

    def prep_slot(t, b):
        for q in range(8):
            gsrc[b, pl.ds(q * 16, 16)] = cb_src[pl.ds(t * SUB + q * 16, 16)]
            gdl[b, pl.ds(q * 16, 16)] = cb_dl[pl.ds(t * SUB + q * 16, 16)]

    def gather_args(t):
        b = t % 2
        return z_hbm.at[gsrc.at[b]], rows.at[b], sems[b]

    def _main(it, rem):
        sup = s + 16 * it
        row0 = pl.multiple_of(sup * (SUPER // SUB), 8)
        pltpu.sync_copy(src_hbm.at[pl.ds(row0, SUPER // SUB)], src_st)
        pltpu.sync_copy(dst_hbm.at[pl.ds(row0, SUPER // SUB)], dst_st)
        base = sup * SUPER
        woffv = jnp.full((16,), rem, jnp.int32)
        for j in range(SUPER // SUB):
            for v in range(SUB // 16):
                off = j * SUB + v * 16
                d = dst_st[j, pl.ds(v * 16, 16)]
                dl = d - cbase
                ok = (dl >= 0) & (dl < HALF) & (base + off + iota16 < E)
                sv = src_st[j, pl.ds(v * 16, 16)]
                cum = plsc.cumsum(ok.astype(jnp.int32))
                pos = woffv + cum - 1
                plsc.store_scatter(cb_dl, [pos], dl, mask=ok)
                plsc.store_scatter(cb_src, [pos], sv, mask=ok)
                woffv = woffv + plsc.all_reduce_population_count(ok)
        woff = lax.squeeze(lax.slice(woffv, (0,), (1,)), dimensions=(0,))
        nfull = lax.div(woff, SUB)

        @pl.when(nfull > 0)
        def _():
            prep_slot(0, 0)
            pltpu.async_copy(*gather_args(0))

        for t in range(8):
            @pl.when(t < nfull)
            def _(t=t):
                pltpu.make_async_copy(*gather_args(t)).wait()

            if t + 1 < 8:
                @pl.when(t + 1 < nfull)
                def _(t=t):
                    prep_slot(t + 1, (t + 1) % 2)
                    pltpu.async_copy(*gather_args(t + 1))

            @pl.when(t < nfull)
            def _(t=t):
                pltpu.sync_copy(rows.at[t % 2], agg_sh.at[gdl.at[t % 2]],
                                add=True)

        shoff = pl.multiple_of(nfull * SUB, 8)
        for q in range(8):
            t1 = cb_src[pl.ds(shoff + q * 16, 16)]
            t2 = cb_dl[pl.ds(shoff + q * 16, 16)]
            cb_src[pl.ds(q * 16, 16)] = t1
            cb_dl[pl.ds(q * 16, 16)] = t2
        return woff - nfull * SUB

    rem = pl.loop(0, SUPER_PER_TILE, init_carry=jnp.int32(0))(_main)

    for q in range(8):
        cb_src[pl.ds(rem + q * 16, 16)] = iota16 + (q * 16)
        cb_dl[pl.ds(rem + q * 16, 16)] = TRASH0 + iota16 + (q * 16)
    prep_slot(0, 0)
    pltpu.async_copy(*gather_args(0))
    pltpu.make_async_copy(*gather_args(0)).wait()
    pltpu.sync_copy(rows.at[0], agg_sh.at[gdl.at[0]], add=True)

    plsc.subcore_barrier()

    @pl.when(s == 0)
    def _():
        cb = pl.multiple_of(cbase, 8)
        pltpu.sync_copy(agg_sh.at[pl.ds(0, HALF)],
                        agg_hbm.at[pl.ds(cb, HALF)])


def _post_body(agg_ref, deg_ref, batch_ref, wg_ref, bg_ref, wo_ref, bo_ref,
               out_ref, acc_ref):
    i = pl.program_id(0)
    deg = jnp.maximum(jnp.sum(deg_ref[0, :, 0, :], axis=0), 1.0)
    h = agg_ref[...] / deg[:, None]
    h = jnp.maximum(
        jnp.dot(h, wg_ref[...], preferred_element_type=jnp.float32)
        + bg_ref[...], 0.0)
    b = batch_ref[0, 0, :]
    oh = (b[:, None] == lax.broadcasted_iota(jnp.int32, (BLK, NUM_GRAPHS), 1))
    hx = jnp.concatenate([h, jnp.ones((BLK, 1), jnp.float32)], axis=1)
    part = lax.dot_general(oh.astype(jnp.float32), hx,
                           (((0,), (0,)), ((), ())),
                           preferred_element_type=jnp.float32)

    @pl.when(i == 0)
    def _():
        acc_ref[...] = part

    @pl.when(i > 0)
    def _():
        acc_ref[...] += part

    @pl.when(i == NBLK - 1)
    def _():
        sums = acc_ref[:, :F]
        cnt = jnp.maximum(acc_ref[:, F:F + 1], 1.0)
        out_ref[...] = (
            jnp.dot(sums / cnt, wo_ref[...],
                    preferred_element_type=jnp.float32)
            + bo_ref[...])


def kernel(type_ids, c, gm, pos, r, vid, edge_index, batch,
           W1, b1, W2, b2, Wg, bg, Wo, bo):
    tid3 = type_ids.astype(jnp.int32).reshape(NBLK_A, 1, BLK_A)
    f3 = [x.astype(jnp.float32).reshape(NBLK_A, 1, BLK_A)
          for x in (c, gm, pos, r, vid)]

    src = jnp.pad(edge_index[0].astype(jnp.int32), (0, PAD_E - E))
    dst = jnp.pad(edge_index[1].astype(jnp.int32), (0, PAD_E - E))
    src2 = src.reshape(EROWS, SUB)
    dst2 = dst.reshape(EROWS, SUB)

    vec3 = pl.BlockSpec((1, 1, BLK_A), lambda i: (i, 0, 0))
    full = lambda a, b: pl.BlockSpec((a, b), lambda i: (0, 0))
    z = pl.pallas_call(
        _feat_body,
        grid=(NBLK_A,),
        in_specs=[vec3, vec3, vec3, vec3, vec3, vec3,
                  full(NUM_TYPES, F // 2), full(1, F // 2),
                  full(5, F // 2), full(1, F // 2)],
        out_specs=pl.BlockSpec((BLK_A, F), lambda i: (i, 0)),
        out_shape=jax.ShapeDtypeStruct((N, F), jnp.float32),
    )(tid3, *f3, W1, b1.reshape(1, F // 2), W2, b2.reshape(1, F // 2))

    mesh = plsc.VectorSubcoreMesh(core_axis_name="c", subcore_axis_name="s")
    cp = pltpu.CompilerParams()
    if "needs_layout_passes" in pltpu.CompilerParams.__dataclass_fields__:
        cp = dataclasses.replace(cp, needs_layout_passes=False)
    if "use_tc_tiling_on_sc" in pltpu.CompilerParams.__dataclass_fields__:
        cp = dataclasses.replace(cp, use_tc_tiling_on_sc=False)
    deg16 = pl.kernel(
        _deg_body,
        out_type=jax.ShapeDtypeStruct((NBLK, 16, 1, BLK), jnp.float32),
        mesh=mesh,
        scratch_types=[
            pltpu.VMEM((SUPER2 // SUB, SUB), jnp.int32),
            pltpu.VMEM((HALF,), jnp.float32),
        ],
        compiler_params=cp,
    )(dst2)

    src2, dst2 = lax.optimization_barrier((src2, dst2, deg16))[:2]

    agg = pl.kernel(
        _edge_body,
        out_type=jax.ShapeDtypeStruct((N, F), jnp.float32),
        mesh=mesh,
        scratch_types=[
            pltpu.VMEM_SHARED((AGG_ROWS, F), jnp.float32),
            pltpu.VMEM((SUPER // SUB, SUB), jnp.int32),
            pltpu.VMEM((SUPER // SUB, SUB), jnp.int32),
            pltpu.VMEM((2, SUB, F), jnp.float32),
            pltpu.VMEM((CAP,), jnp.int32),
            pltpu.VMEM((CAP,), jnp.int32),
            pltpu.VMEM((2, SUB), jnp.int32),
            pltpu.VMEM((2, SUB), jnp.int32),
            pltpu.VMEM((SUB, F), jnp.float32),
            pltpu.SemaphoreType.DMA,
            pltpu.SemaphoreType.DMA,
        ],
        compiler_params=cp,
    )(z, src2, dst2)

    batch3 = batch.astype(jnp.int32).reshape(NBLK, 1, BLK)
    pred = pl.pallas_call(
        _post_body,
        grid=(NBLK,),
        in_specs=[
            pl.BlockSpec((BLK, F), lambda i: (i, 0)),
            pl.BlockSpec((1, 16, 1, BLK), lambda i: (i, 0, 0, 0)),
            pl.BlockSpec((1, 1, BLK), lambda i: (i, 0, 0)),
            pl.BlockSpec((F, F), lambda i: (0, 0)),
            pl.BlockSpec((1, F), lambda i: (0, 0)),
            pl.BlockSpec((F, 4), lambda i: (0, 0)),
            pl.BlockSpec((1, 4), lambda i: (0, 0)),
        ],
        out_specs=pl.BlockSpec((NUM_GRAPHS, 4), lambda i: (0, 0)),
        out_shape=jax.ShapeDtypeStruct((NUM_GRAPHS, 4), jnp.float32),
        scratch_shapes=[pltpu.VMEM((NUM_GRAPHS, F + 1), jnp.float32)],
    )(agg, deg16, batch3, Wg, bg.reshape(1, F), Wo, bo.reshape(1, 4))

    return pred


# speedup vs baseline: 13.0163x; 1.0248x over previous
import dataclasses
import functools

import jax
import jax.numpy as jnp
from jax import lax
from jax.experimental import pallas as pl
from jax.experimental.pallas import tpu as pltpu
from jax.experimental.pallas import tpu_sc as plsc

N = 100000
E = 1600000
F = 32
NUM_TYPES = 25
NUM_GRAPHS = 256
HALF = N // 2

BLK = 2000
NBLK = N // BLK
BLK_A = 10000
NBLK_A = N // BLK_A

SUB = 128
SUPER = 2048
N_SUPER = (E + SUPER - 1) // SUPER
SUPER_PER_TILE = (N_SUPER + 15) // 16
PAD_E = 1605632
EROWS = PAD_E // SUB
NFULL_MAX = (SUPER + SUB - 1) // SUB
CAP = SUPER + SUB
SUPER2 = 4096
N_SUPER2 = (E + SUPER2 - 1) // SUPER2
SUPER2_PER_TILE = (N_SUPER2 + 15) // 16

AGG_ROWS = 51200
TRASH0 = 50048
ZSTRIPE = AGG_ROWS // 16
WSTRIPE = HALF // 16
HSTRIPE = HALF // 16


def _feat_body(tid_ref, c_ref, gm_ref, pos_ref, r_ref, vid_ref,
               w1_ref, b1_ref, w2_ref, b2_ref, z_ref):
    t = tid_ref[0, 0, :]
    oh = (t[:, None] == lax.broadcasted_iota(jnp.int32, (BLK_A, NUM_TYPES), 1))
    xt = (jnp.dot(oh.astype(jnp.float32), w1_ref[...],
                  preferred_element_type=jnp.float32) + b1_ref[...])
    w2 = w2_ref[...]
    xn = (b2_ref[...]
          + c_ref[0, 0, :][:, None] * w2[0:1, :]
          + gm_ref[0, 0, :][:, None] * w2[1:2, :]
          + pos_ref[0, 0, :][:, None] * w2[2:3, :]
          + r_ref[0, 0, :][:, None] * w2[3:4, :]
          + vid_ref[0, 0, :][:, None] * w2[4:5, :])
    z_ref[...] = jnp.concatenate([xt, xn], axis=1)


def _deg_body(dst_hbm, deg_hbm, dst_st, hist):
    c = lax.axis_index("c")
    s = lax.axis_index("s")
    cbase = c * HALF
    ones16 = jnp.ones((16,), jnp.float32)

    @pl.loop(0, HALF, step=16)
    def _(i):
        hist[pl.ds(i, 16)] = jnp.zeros((16,), jnp.float32)

    @pl.loop(0, SUPER2_PER_TILE)
    def _(it):
        sup = s + 16 * it

        @pl.when(sup < N_SUPER2)
        def _():
            row0 = pl.multiple_of(sup * (SUPER2 // SUB), 8)
            pltpu.sync_copy(dst_hbm.at[pl.ds(row0, SUPER2 // SUB)], dst_st)
            base = sup * SUPER2
            for j in range(SUPER2 // SUB):
                @pl.when(base + j * SUB < E)
                def _(j=j):
                    for v in range(SUB // 16):
                        d = dst_st[j, pl.ds(v * 16, 16)]
                        dl = d - cbase
                        ok = (dl >= 0) & (dl < HALF)
                        plsc.addupdate_scatter(
                            hist, [jnp.where(ok, dl, 0)], ones16, mask=ok)

    for k in range(HALF // BLK):
        pltpu.sync_copy(hist.at[pl.ds(k * BLK, BLK)],
                        deg_hbm.at[c * (HALF // BLK) + k, s, 0])


def _edge_body(z_hbm, src_hbm, dst_hbm, agg_hbm,
               agg_sh, src_st, dst_st, rows, cb_src, cb_dl, gsrc, gdl,
               zeros_blk, sem_a, sem_b):
    c = lax.axis_index("c")
    s = lax.axis_index("s")
    cbase = c * HALF

    @pl.loop(0, SUB)
    def _(r):
        zeros_blk[r, pl.ds(0, 16)] = jnp.zeros((16,), jnp.float32)
        zeros_blk[r, pl.ds(16, 16)] = jnp.zeros((16,), jnp.float32)

    zoff = pl.multiple_of(s * ZSTRIPE, 8)
    for k in range(ZSTRIPE // SUB):
        pltpu.sync_copy(zeros_blk,
                        agg_sh.at[pl.ds(zoff + k * SUB, SUB)])

    plsc.subcore_barrier()

    iota16 = lax.iota(jnp.int32, 16)
    sems = (sem_a, sem_b)

    def prep_slot(t, b):
        for q in range(8):
            gsrc[b, pl.ds(q * 16, 16)] = cb_src[pl.ds(t * SUB + q * 16, 16)]
            gdl[b, pl.ds(q * 16, 16)] = cb_dl[pl.ds(t * SUB + q * 16, 16)]

    def gather_args(t):
        b = t % 2
        return z_hbm.at[gsrc.at[b]], rows.at[b], sems[b]

    def _main(it, rem):
        sup = s + 16 * it
        row0 = pl.multiple_of(sup * (SUPER // SUB), 8)
        pltpu.sync_copy(src_hbm.at[pl.ds(row0, SUPER // SUB)], src_st)
        pltpu.sync_copy(dst_hbm.at[pl.ds(row0, SUPER // SUB)], dst_st)
        base = sup * SUPER
        woffv = jnp.full((16,), rem, jnp.int32)
        for j in range(SUPER // SUB):
            for v in range(SUB // 16):
                off = j * SUB + v * 16
                d = dst_st[j, pl.ds(v * 16, 16)]
                dl = d - cbase
                ok = (dl >= 0) & (dl < HALF) & (base + off + iota16 < E)
                sv = src_st[j, pl.ds(v * 16, 16)]
                cum = plsc.cumsum(ok.astype(jnp.int32))
                pos = woffv + cum - 1
                plsc.store_scatter(cb_dl, [pos], dl, mask=ok)
                plsc.store_scatter(cb_src, [pos], sv, mask=ok)
                woffv = woffv + plsc.all_reduce_population_count(ok)
        woff = lax.squeeze(lax.slice(woffv, (0,), (1,)), dimensions=(0,))
        nfull = lax.div(woff, SUB)

        @pl.when(nfull > 0)
        def _():
            prep_slot(0, 0)
            pltpu.async_copy(*gather_args(0))

        for t in range(NFULL_MAX):
            @pl.when(t < nfull)
            def _(t=t):
                pltpu.make_async_copy(*gather_args(t)).wait()

            if t + 1 < NFULL_MAX:
                @pl.when(t + 1 < nfull)
                def _(t=t):
                    prep_slot(t + 1, (t + 1) % 2)
                    pltpu.async_copy(*gather_args(t + 1))

            @pl.when(t < nfull)
            def _(t=t):
                pltpu.sync_copy(rows.at[t % 2], agg_sh.at[gdl.at[t % 2]],
                                add=True)

        shoff = pl.multiple_of(nfull * SUB, 8)
        for q in range(8):
            t1 = cb_src[pl.ds(shoff + q * 16, 16)]
            t2 = cb_dl[pl.ds(shoff + q * 16, 16)]
            cb_src[pl.ds(q * 16, 16)] = t1
            cb_dl[pl.ds(q * 16, 16)] = t2
        return woff - nfull * SUB

    rem = pl.loop(0, SUPER_PER_TILE, init_carry=jnp.int32(0))(_main)

    for q in range(8):
        cb_src[pl.ds(rem + q * 16, 16)] = iota16 + (q * 16)
        cb_dl[pl.ds(rem + q * 16, 16)] = TRASH0 + iota16 + (q * 16)
    prep_slot(0, 0)
    pltpu.async_copy(*gather_args(0))
    pltpu.make_async_copy(*gather_args(0)).wait()
    pltpu.sync_copy(rows.at[0], agg_sh.at[gdl.at[0]], add=True)

    plsc.subcore_barrier()

    @pl.when(s == 0)
    def _():
        cb = pl.multiple_of(cbase, 8)
        pltpu.sync_copy(agg_sh.at[pl.ds(0, HALF)],
                        agg_hbm.at[pl.ds(cb, HALF)])


def _post_body(agg_ref, deg_ref, batch_ref, wg_ref, bg_ref, wo_ref, bo_ref,
               out_ref, acc_ref):
    i = pl.program_id(0)
    deg = jnp.maximum(jnp.sum(deg_ref[0, :, 0, :], axis=0), 1.0)
    h = agg_ref[...] / deg[:, None]
    h = jnp.maximum(
        jnp.dot(h, wg_ref[...], preferred_element_type=jnp.float32)
        + bg_ref[...], 0.0)
    b = batch_ref[0, 0, :]
    oh = (b[:, None] == lax.broadcasted_iota(jnp.int32, (BLK, NUM_GRAPHS), 1))
    hx = jnp.concatenate([h, jnp.ones((BLK, 1), jnp.float32)], axis=1)
    part = lax.dot_general(oh.astype(jnp.float32), hx,
                           (((0,), (0,)), ((), ())),
                           preferred_element_type=jnp.float32)

    @pl.when(i == 0)
    def _():
        acc_ref[...] = part

    @pl.when(i > 0)
    def _():
        acc_ref[...] += part

    @pl.when(i == NBLK - 1)
    def _():
        sums = acc_ref[:, :F]
        cnt = jnp.maximum(acc_ref[:, F:F + 1], 1.0)
        out_ref[...] = (
            jnp.dot(sums / cnt, wo_ref[...],
                    preferred_element_type=jnp.float32)
            + bo_ref[...])


def kernel(type_ids, c, gm, pos, r, vid, edge_index, batch,
           W1, b1, W2, b2, Wg, bg, Wo, bo):
    tid3 = type_ids.astype(jnp.int32).reshape(NBLK_A, 1, BLK_A)
    f3 = [x.astype(jnp.float32).reshape(NBLK_A, 1, BLK_A)
          for x in (c, gm, pos, r, vid)]

    src = jnp.pad(edge_index[0].astype(jnp.int32), (0, PAD_E - E))
    dst = jnp.pad(edge_index[1].astype(jnp.int32), (0, PAD_E - E))
    src2 = src.reshape(EROWS, SUB)
    dst2 = dst.reshape(EROWS, SUB)

    vec3 = pl.BlockSpec((1, 1, BLK_A), lambda i: (i, 0, 0))
    full = lambda a, b: pl.BlockSpec((a, b), lambda i: (0, 0))
    z = pl.pallas_call(
        _feat_body,
        grid=(NBLK_A,),
        in_specs=[vec3, vec3, vec3, vec3, vec3, vec3,
                  full(NUM_TYPES, F // 2), full(1, F // 2),
                  full(5, F // 2), full(1, F // 2)],
        out_specs=pl.BlockSpec((BLK_A, F), lambda i: (i, 0)),
        out_shape=jax.ShapeDtypeStruct((N, F), jnp.float32),
    )(tid3, *f3, W1, b1.reshape(1, F // 2), W2, b2.reshape(1, F // 2))

    mesh = plsc.VectorSubcoreMesh(core_axis_name="c", subcore_axis_name="s")
    cp = pltpu.CompilerParams()
    if "needs_layout_passes" in pltpu.CompilerParams.__dataclass_fields__:
        cp = dataclasses.replace(cp, needs_layout_passes=False)
    if "use_tc_tiling_on_sc" in pltpu.CompilerParams.__dataclass_fields__:
        cp = dataclasses.replace(cp, use_tc_tiling_on_sc=False)
    deg16 = pl.kernel(
        _deg_body,
        out_type=jax.ShapeDtypeStruct((NBLK, 16, 1, BLK), jnp.float32),
        mesh=mesh,
        scratch_types=[
            pltpu.VMEM((SUPER2 // SUB, SUB), jnp.int32),
            pltpu.VMEM((HALF,), jnp.float32),
        ],
        compiler_params=cp,
    )(dst2)

    src2, dst2 = lax.optimization_barrier((src2, dst2, deg16))[:2]

    agg = pl.kernel(
        _edge_body,
        out_type=jax.ShapeDtypeStruct((N, F), jnp.float32),
        mesh=mesh,
        scratch_types=[
            pltpu.VMEM_SHARED((AGG_ROWS, F), jnp.float32),
            pltpu.VMEM((SUPER // SUB, SUB), jnp.int32),
            pltpu.VMEM((SUPER // SUB, SUB), jnp.int32),
            pltpu.VMEM((2, SUB, F), jnp.float32),
            pltpu.VMEM((CAP,), jnp.int32),
            pltpu.VMEM((CAP,), jnp.int32),
            pltpu.VMEM((2, SUB), jnp.int32),
            pltpu.VMEM((2, SUB), jnp.int32),
            pltpu.VMEM((SUB, F), jnp.float32),
            pltpu.SemaphoreType.DMA,
            pltpu.SemaphoreType.DMA,
        ],
        compiler_params=cp,
    )(z, src2, dst2)

    batch3 = batch.astype(jnp.int32).reshape(NBLK, 1, BLK)
    pred = pl.pallas_call(
        _post_body,
        grid=(NBLK,),
        in_specs=[
            pl.BlockSpec((BLK, F), lambda i: (i, 0)),
            pl.BlockSpec((1, 16, 1, BLK), lambda i: (i, 0, 0, 0)),
            pl.BlockSpec((1, 1, BLK), lambda i: (i, 0, 0)),
            pl.BlockSpec((F, F), lambda i: (0, 0)),
            pl.BlockSpec((1, F), lambda i: (0, 0)),
            pl.BlockSpec((F, 4), lambda i: (0, 0)),
            pl.BlockSpec((1, 4), lambda i: (0, 0)),
        ],
        out_specs=pl.BlockSpec((NUM_GRAPHS, 4), lambda i: (0, 0)),
        out_shape=jax.ShapeDtypeStruct((NUM_GRAPHS, 4), jnp.float32),
        scratch_shapes=[pltpu.VMEM((NUM_GRAPHS, F + 1), jnp.float32)],
    )(agg, deg16, batch3, Wg, bg.reshape(1, F), Wo, bo.reshape(1, 4))

    return pred
